# baseline (speedup 1.0000x reference)
import functools

import jax
import jax.numpy as jnp
from jax import lax
from jax.experimental import pallas as pl
from jax.experimental.pallas import tpu as pltpu
from jax.experimental.pallas import tpu_sc as plsc

N_ROWS = 1_000_000
DIM = 32
BATCH = 16384

_info = plsc.get_sparse_core_info()
_NC, _NS = _info.num_cores, _info.num_subcores
_NW = _NC * _NS
_B_PER_W = BATCH // _NW


def _gather_body(x_hbm, table_hbm, out_hbm, idx_v, rows_v, sem):
    wid = lax.axis_index("s") * _NC + lax.axis_index("c")
    base = wid * _B_PER_W
    pltpu.sync_copy(x_hbm.at[pl.ds(base, _B_PER_W)], idx_v)
    pltpu.async_copy(table_hbm.at[idx_v], rows_v, sem).wait()
    pltpu.sync_copy(rows_v, out_hbm.at[pl.ds(base, _B_PER_W)])


@jax.jit
def _gather(x, table):
    mesh = plsc.VectorSubcoreMesh(core_axis_name="c", subcore_axis_name="s")
    kern = functools.partial(
        pl.kernel,
        mesh=mesh,
        out_type=jax.ShapeDtypeStruct((BATCH, DIM), jnp.float32),
        scratch_types=[
            pltpu.VMEM((_B_PER_W,), jnp.int32),
            pltpu.VMEM((_B_PER_W, DIM), jnp.float32),
            pltpu.SemaphoreType.DMA,
        ],
        compiler_params=pltpu.CompilerParams(use_tc_tiling_on_sc=False),
    )(_gather_body)
    return kern(x, table)


def kernel(x, table):
    return _gather(x, table)


# speedup vs baseline: 1.6608x; 1.6608x over previous
import functools

import jax
import jax.numpy as jnp
from jax import lax
from jax.experimental import pallas as pl
from jax.experimental.pallas import tpu as pltpu
from jax.experimental.pallas import tpu_sc as plsc

N_ROWS = 1_000_000
DIM = 32
BATCH = 16384

_info = plsc.get_sparse_core_info()
_NC, _NS = _info.num_cores, _info.num_subcores
_NW = _NC * _NS
_B_PER_W = BATCH // _NW


def _gather_body(x_hbm, table_hbm, out_hbm, idx_s, rows_v, sem):
    wid = lax.axis_index("s") * _NC + lax.axis_index("c")
    base = wid * _B_PER_W
    pltpu.sync_copy(x_hbm.at[pl.ds(base, _B_PER_W)], idx_s)

    def _issue(j, _):
        idx16 = idx_s[pl.ds(j * 16, 16)]
        for l in range(16):
            pltpu.async_copy(table_hbm.at[idx16[l]], rows_v.at[j * 16 + l], sem)
        return ()

    lax.fori_loop(0, _B_PER_W // 16, _issue, ())

    def _drain(j, _):
        pltpu.make_async_copy(table_hbm.at[0], rows_v.at[j], sem).wait()
        return ()

    lax.fori_loop(0, _B_PER_W, _drain, ())
    pltpu.sync_copy(rows_v, out_hbm.at[pl.ds(base, _B_PER_W)])


@jax.jit
def _gather(x, table):
    mesh = plsc.VectorSubcoreMesh(core_axis_name="c", subcore_axis_name="s")
    kern = functools.partial(
        pl.kernel,
        mesh=mesh,
        out_type=jax.ShapeDtypeStruct((BATCH, DIM), jnp.float32),
        scratch_types=[
            pltpu.VMEM((_B_PER_W,), jnp.int32),
            pltpu.VMEM((_B_PER_W, DIM), jnp.float32),
            pltpu.SemaphoreType.DMA,
        ],
    )(_gather_body)
    return kern(x, table)


def kernel(x, table):
    return _gather(x, table)


# speedup vs baseline: 4.6837x; 2.8201x over previous
import functools

import jax
import jax.numpy as jnp
from jax import lax
from jax.experimental import pallas as pl
from jax.experimental.pallas import tpu as pltpu
from jax.experimental.pallas import tpu_sc as plsc

N_ROWS = 1_000_000
DIM = 32
BATCH = 16384

_info = plsc.get_sparse_core_info()
_NC, _NS = _info.num_cores, _info.num_subcores
_NW = _NC * _NS
_B_PER_W = BATCH // _NW
_G = 8
_NGROUPS = _B_PER_W // _G


def _gather_body(x_hbm, tableT_hbm, outT_hbm, idx_v, blocks_v, cols_v, sem):
    wid = lax.axis_index("s") * _NC + lax.axis_index("c")
    base = wid * _B_PER_W
    pltpu.sync_copy(x_hbm.at[pl.ds(base, _B_PER_W)], idx_v.at[pl.ds(0, _B_PER_W)])

    iota = lax.broadcasted_iota(jnp.int32, (16,), 0)

    def _fire(idx16, l, bank):
        i = idx16[l]
        col0 = pl.multiple_of((i // 128) * 128, 128)
        pltpu.async_copy(
            tableT_hbm.at[:, pl.ds(col0, 128)],
            blocks_v.at[bank, l],
            sem,
        )

    idx16_0 = idx_v[pl.ds(0, 16)]
    for l in range(_G):
        _fire(idx16_0, l, 0)

    def _step(g, carry):
        (idx16_cur,) = carry
        bank = lax.rem(g, 2)
        nxt = 1 - bank
        idx16_next = idx_v[pl.ds((g + 1) * _G, 16)]

        @pl.when(g < _NGROUPS - 1)
        def _():
            for l in range(_G):
                _fire(idx16_next, l, nxt)

        for l in range(_G):
            pltpu.make_async_copy(
                tableT_hbm.at[:, pl.ds(0, 128)],
                blocks_v.at[bank, l],
                sem,
            ).wait()

        for l in range(_G):
            r = lax.rem(idx16_cur[l], 128)
            rvec = lax.broadcast(r, (16,))
            b = g * _G + l
            bvec = lax.broadcast(b, (16,))
            for h in range(2):
                rows = iota + 16 * h
                vals = plsc.load_gather(blocks_v.at[bank, l], [rows, rvec])
                plsc.store_scatter(cols_v, [rows, bvec], vals)
        return (idx16_next,)

    lax.fori_loop(0, _NGROUPS, _step, (idx16_0,), unroll=False)
    pltpu.sync_copy(cols_v, outT_hbm.at[:, pl.ds(base, _B_PER_W)])


@jax.jit
def _gather(x, tableT):
    mesh = plsc.VectorSubcoreMesh(core_axis_name="c", subcore_axis_name="s")
    kern = functools.partial(
        pl.kernel,
        mesh=mesh,
        out_type=jax.ShapeDtypeStruct((DIM, BATCH), jnp.float32),
        scratch_types=[
            pltpu.VMEM((_B_PER_W + 16,), jnp.int32),
            pltpu.VMEM((2, _G, DIM, 128), jnp.float32),
            pltpu.VMEM((DIM, _B_PER_W), jnp.float32),
            pltpu.SemaphoreType.DMA,
        ],
        compiler_params=pltpu.CompilerParams(needs_layout_passes=False),
    )(_gather_body)
    return kern(x, tableT)


def kernel(x, table):
    return _gather(x, table.T).T
